# baseline (device time: 527035 ns/iter reference)
import jax
import jax.numpy as jnp
from jax import lax
from jax.experimental import pallas as pl
from jax.experimental.pallas import tpu as pltpu

N_DEV = 16
B, SQ, D_MODEL = 2, 512, 768
HQ, DH = 8, 64
SKV_LOCAL = 512
D_HEADS = HQ * DH
ACC_COLS = D_HEADS + 128
ACC_ROWS = B * SQ
N_HOPS = N_DEV - 1


def kernel(x, Wq, K_ext, V_ext, Wo):
    def body(x_ref, wq_ref, k_ref, v_ref, wo_ref, out_ref,
             acc_ref, comm_ref, send_sems, recv_sems, credit_sem):
        my_pos = lax.axis_index("i")
        left = lax.rem(my_pos - 1 + N_DEV, N_DEV)
        right = lax.rem(my_pos + 1, N_DEV)

        barrier_sem = pltpu.get_barrier_semaphore()
        for nbr in (left, right):
            pl.semaphore_signal(barrier_sem, inc=1, device_id=(nbr,),
                                device_id_type=pl.DeviceIdType.MESH)
        pl.semaphore_wait(barrier_sem, 2)

        row_iota = lax.broadcasted_iota(jnp.int32, (SQ, SKV_LOCAL), 0)
        col_iota = lax.broadcasted_iota(jnp.int32, (SQ, SKV_LOCAL), 1)
        qb = row_iota // 64
        kb = my_pos * (SKV_LOCAL // 64) + col_iota // 64
        mask = (qb == kb) | (kb == 0) | (lax.rem(qb + kb, 3) == 0)

        acc_ref[:, :] = jnp.zeros((ACC_ROWS, ACC_COLS), jnp.float32)

        for b in range(B):
            q_b = jnp.dot(x_ref[b], wq_ref[:, :],
                          preferred_element_type=jnp.float32)
            for h in range(HQ):
                q_h = q_b[:, h * DH:(h + 1) * DH]
                k_h = k_ref[b, :, h, :]
                v_h = v_ref[b, :, h, :]
                s = lax.dot_general(
                    q_h, k_h, (((1,), (1,)), ((), ())),
                    preferred_element_type=jnp.float32) * 0.125
                w = jnp.where(mask, jnp.exp(s), 0.0)
                l = jnp.sum(w, axis=1, keepdims=True)
                ctx = jnp.dot(w, v_h, preferred_element_type=jnp.float32)
                r0 = b * SQ
                acc_ref[r0:r0 + SQ, h * DH:(h + 1) * DH] = ctx
                acc_ref[r0:r0 + SQ, D_HEADS + h:D_HEADS + h + 1] = l

        comm_ref[0, :, :] = acc_ref[:, :]

        for hop in range(N_HOPS):
            send_slot = hop % 2
            recv_slot = (hop + 1) % 2
            if hop > 0:
                pl.semaphore_wait(credit_sem, 1)
            rdma = pltpu.make_async_remote_copy(
                src_ref=comm_ref.at[send_slot],
                dst_ref=comm_ref.at[recv_slot],
                send_sem=send_sems.at[send_slot],
                recv_sem=recv_sems.at[recv_slot],
                device_id=(right,),
                device_id_type=pl.DeviceIdType.MESH,
            )
            rdma.start()
            rdma.wait()
            acc_ref[:, :] = acc_ref[:, :] + comm_ref[recv_slot]
            if hop < N_HOPS - 1:
                pl.semaphore_signal(credit_sem, inc=1, device_id=(left,),
                                    device_id_type=pl.DeviceIdType.MESH)

        for b in range(B):
            r0 = b * SQ
            parts = []
            for h in range(HQ):
                ctx_h = acc_ref[r0:r0 + SQ, h * DH:(h + 1) * DH]
                l_h = acc_ref[r0:r0 + SQ, D_HEADS + h:D_HEADS + h + 1]
                parts.append(ctx_h / l_h)
            ctx_n = jnp.concatenate(parts, axis=1)
            out_ref[b] = jnp.dot(ctx_n, wo_ref[:, :],
                                 preferred_element_type=jnp.float32)

    return pl.pallas_call(
        body,
        out_shape=jax.ShapeDtypeStruct((B, SQ, D_MODEL), jnp.float32),
        in_specs=[pl.BlockSpec(memory_space=pltpu.VMEM)] * 5,
        out_specs=pl.BlockSpec(memory_space=pltpu.VMEM),
        scratch_shapes=[
            pltpu.VMEM((ACC_ROWS, ACC_COLS), jnp.float32),
            pltpu.VMEM((2, ACC_ROWS, ACC_COLS), jnp.float32),
            pltpu.SemaphoreType.DMA((2,)),
            pltpu.SemaphoreType.DMA((2,)),
            pltpu.SemaphoreType.REGULAR,
        ],
        compiler_params=pltpu.CompilerParams(collective_id=0),
    )(x, Wq, K_ext, V_ext, Wo)


# device time: 95544 ns/iter; 5.5161x vs baseline; 5.5161x over previous
import jax
import jax.numpy as jnp
from jax import lax
from jax.experimental import pallas as pl
from jax.experimental.pallas import tpu as pltpu

N_DEV = 16
B, SQ, D_MODEL = 2, 512, 768
HQ, DH = 8, 64
SKV_LOCAL = 512
D_HEADS = HQ * DH
ACC_COLS = D_HEADS + 128
ACC_ROWS = B * SQ
RS_RECV_OFF = (0, 512, 768, 896)


def kernel(x, Wq, K_ext, V_ext, Wo):
    def body(x_ref, wq_ref, k_ref, v_ref, wo_ref, out_ref,
             acc_ref, rs_recv_ref, send_sems, recv_sems):
        my_pos = lax.axis_index("i")

        barrier_sem = pltpu.get_barrier_semaphore()
        for k in range(4):
            pl.semaphore_signal(barrier_sem, inc=1,
                                device_id=(my_pos ^ (1 << k),),
                                device_id_type=pl.DeviceIdType.MESH)
        pl.semaphore_wait(barrier_sem, 4)

        row_iota = lax.broadcasted_iota(jnp.int32, (SQ, SKV_LOCAL), 0)
        col_iota = lax.broadcasted_iota(jnp.int32, (SQ, SKV_LOCAL), 1)
        qb = row_iota // 64
        kb = my_pos * (SKV_LOCAL // 64) + col_iota // 64
        mask = (qb == kb) | (kb == 0) | (lax.rem(qb + kb, 3) == 0)

        acc_ref[:, :] = jnp.zeros((ACC_ROWS, ACC_COLS), jnp.float32)

        for b in range(B):
            q_b = jnp.dot(x_ref[b], wq_ref[:, :],
                          preferred_element_type=jnp.float32)
            for h in range(HQ):
                q_h = q_b[:, h * DH:(h + 1) * DH]
                k_h = k_ref[b, :, h, :]
                v_h = v_ref[b, :, h, :]
                s = lax.dot_general(
                    q_h, k_h, (((1,), (1,)), ((), ())),
                    preferred_element_type=jnp.float32) * 0.125
                w = jnp.where(mask, jnp.exp(s), 0.0)
                l = jnp.sum(w, axis=1, keepdims=True)
                ctx = jnp.dot(w, v_h, preferred_element_type=jnp.float32)
                r0 = b * SQ
                acc_ref[r0:r0 + SQ, h * DH:(h + 1) * DH] = ctx
                acc_ref[r0:r0 + SQ, D_HEADS + h:D_HEADS + h + 1] = l

        w_off = jnp.int32(0)
        for k in range(4):
            half = 512 >> k
            pk = (my_pos >> k) & 1
            send_off = w_off + (1 - pk) * half
            keep_off = w_off + pk * half
            ro = RS_RECV_OFF[k]
            rdma = pltpu.make_async_remote_copy(
                src_ref=acc_ref.at[pl.ds(send_off, half), :],
                dst_ref=rs_recv_ref.at[ro:ro + half, :],
                send_sem=send_sems.at[k],
                recv_sem=recv_sems.at[k],
                device_id=(my_pos ^ (1 << k),),
                device_id_type=pl.DeviceIdType.MESH,
            )
            rdma.start()
            rdma.wait()
            acc_ref[pl.ds(keep_off, half), :] = (
                acc_ref[pl.ds(keep_off, half), :]
                + rs_recv_ref[ro:ro + half, :]
            )
            w_off = keep_off

        a_off = w_off
        for j in range(4):
            k = 3 - j
            m = 64 << j
            pk = (my_pos >> k) & 1
            rdma = pltpu.make_async_remote_copy(
                src_ref=acc_ref.at[pl.ds(a_off, m), :],
                dst_ref=acc_ref.at[pl.ds(a_off, m), :],
                send_sem=send_sems.at[4 + j],
                recv_sem=recv_sems.at[4 + j],
                device_id=(my_pos ^ (1 << k),),
                device_id_type=pl.DeviceIdType.MESH,
            )
            rdma.start()
            rdma.wait()
            a_off = a_off - pk * m

        for b in range(B):
            r0 = b * SQ
            parts = []
            for h in range(HQ):
                ctx_h = acc_ref[r0:r0 + SQ, h * DH:(h + 1) * DH]
                l_h = acc_ref[r0:r0 + SQ, D_HEADS + h:D_HEADS + h + 1]
                parts.append(ctx_h / l_h)
            ctx_n = jnp.concatenate(parts, axis=1)
            out_ref[b] = jnp.dot(ctx_n, wo_ref[:, :],
                                 preferred_element_type=jnp.float32)

    return pl.pallas_call(
        body,
        out_shape=jax.ShapeDtypeStruct((B, SQ, D_MODEL), jnp.float32),
        in_specs=[pl.BlockSpec(memory_space=pltpu.VMEM)] * 5,
        out_specs=pl.BlockSpec(memory_space=pltpu.VMEM),
        scratch_shapes=[
            pltpu.VMEM((ACC_ROWS, ACC_COLS), jnp.float32),
            pltpu.VMEM((960, ACC_COLS), jnp.float32),
            pltpu.SemaphoreType.DMA((8,)),
            pltpu.SemaphoreType.DMA((8,)),
        ],
        compiler_params=pltpu.CompilerParams(collective_id=0),
    )(x, Wq, K_ext, V_ext, Wo)


# device time: 64062 ns/iter; 8.2270x vs baseline; 1.4914x over previous
import jax
import jax.numpy as jnp
from jax import lax
from jax.experimental import pallas as pl
from jax.experimental.pallas import tpu as pltpu

N_DEV = 16
B, SQ, D_MODEL = 2, 512, 768
HQ, DH = 8, 64
SKV_LOCAL = 512
D_HEADS = HQ * DH
ACC_COLS = D_HEADS + 128
ACC_ROWS = B * SQ
RS_RECV_OFF = (0, 512, 768, 896)
AG_RECV_OFF = (0, 64, 192, 448)


def kernel(x, Wq, K_ext, V_ext, Wo):
    def body(x_ref, wq_ref, k_ref, v_ref, wo_ref, out_ref,
             acc_ref, rs_stage_ref, rs_recv_ref, ag_stage_ref, ag_recv_ref,
             send_sems, recv_sems):
        my_pos = lax.axis_index("i")

        barrier_sem = pltpu.get_barrier_semaphore()
        for k in range(4):
            pl.semaphore_signal(barrier_sem, inc=1,
                                device_id=(my_pos ^ (1 << k),),
                                device_id_type=pl.DeviceIdType.MESH)
        pl.semaphore_wait(barrier_sem, 4)

        row_iota = lax.broadcasted_iota(jnp.int32, (SQ, SKV_LOCAL), 0)
        col_iota = lax.broadcasted_iota(jnp.int32, (SQ, SKV_LOCAL), 1)
        qb = row_iota // 64
        kb = my_pos * (SKV_LOCAL // 64) + col_iota // 64
        mask = (qb == kb) | (kb == 0) | (lax.rem(qb + kb, 3) == 0)

        acc_ref[:, :] = jnp.zeros((ACC_ROWS, ACC_COLS), jnp.float32)

        for b in range(B):
            q_b = jnp.dot(x_ref[b], wq_ref[:, :],
                          preferred_element_type=jnp.float32)
            for h in range(HQ):
                q_h = q_b[:, h * DH:(h + 1) * DH]
                k_h = k_ref[b, :, h, :]
                v_h = v_ref[b, :, h, :]
                s = lax.dot_general(
                    q_h, k_h, (((1,), (1,)), ((), ())),
                    preferred_element_type=jnp.float32) * 0.125
                w = jnp.where(mask, jnp.exp(s), 0.0)
                l = jnp.sum(w, axis=1, keepdims=True)
                ctx = jnp.dot(w, v_h, preferred_element_type=jnp.float32)
                r0 = b * SQ
                acc_ref[r0:r0 + SQ, h * DH:(h + 1) * DH] = ctx
                acc_ref[r0:r0 + SQ, D_HEADS + h:D_HEADS + h + 1] = l

        w_off = jnp.int32(0)
        for k in range(4):
            half = 512 >> k
            pk = (my_pos >> k) & 1
            send_off = w_off + (1 - pk) * half
            keep_off = w_off + pk * half
            ro = RS_RECV_OFF[k]
            rs_stage_ref[0:half, :] = acc_ref[
                pl.ds(send_off, half), :].astype(jnp.bfloat16)
            rdma = pltpu.make_async_remote_copy(
                src_ref=rs_stage_ref.at[0:half, :],
                dst_ref=rs_recv_ref.at[ro:ro + half, :],
                send_sem=send_sems.at[k],
                recv_sem=recv_sems.at[k],
                device_id=(my_pos ^ (1 << k),),
                device_id_type=pl.DeviceIdType.MESH,
            )
            rdma.start()
            rdma.wait()
            acc_ref[pl.ds(keep_off, half), :] = (
                acc_ref[pl.ds(keep_off, half), :]
                + rs_recv_ref[ro:ro + half, :].astype(jnp.float32)
            )
            w_off = keep_off

        for h in range(HQ):
            ctx_h = acc_ref[pl.ds(w_off, 64), h * DH:(h + 1) * DH]
            l_h = acc_ref[pl.ds(w_off, 64), D_HEADS + h:D_HEADS + h + 1]
            acc_ref[pl.ds(w_off, 64), h * DH:(h + 1) * DH] = ctx_h / l_h

        a_off = w_off
        for j in range(4):
            k = 3 - j
            m = 64 << j
            pk = (my_pos >> k) & 1
            ao = AG_RECV_OFF[j]
            ag_stage_ref[0:m, :] = acc_ref[
                pl.ds(a_off, m), 0:D_HEADS].astype(jnp.bfloat16)
            rdma = pltpu.make_async_remote_copy(
                src_ref=ag_stage_ref.at[0:m, :],
                dst_ref=ag_recv_ref.at[ao:ao + m, :],
                send_sem=send_sems.at[4 + j],
                recv_sem=recv_sems.at[4 + j],
                device_id=(my_pos ^ (1 << k),),
                device_id_type=pl.DeviceIdType.MESH,
            )
            rdma.start()

            if j == 3:
                p0 = my_pos & 1

                def compute_out(b):
                    out_ref[b] = jnp.dot(
                        acc_ref[b * SQ:(b + 1) * SQ, 0:D_HEADS],
                        wo_ref[:, :], preferred_element_type=jnp.float32)

                @pl.when(p0 == 0)
                def _():
                    compute_out(0)

                @pl.when(p0 == 1)
                def _():
                    compute_out(1)

            rdma.wait()
            sib_off = a_off + m - 2 * pk * m
            acc_ref[pl.ds(sib_off, m), 0:D_HEADS] = ag_recv_ref[
                ao:ao + m, :].astype(jnp.float32)
            a_off = a_off - pk * m

            if j == 3:
                @pl.when(p0 == 0)
                def _():
                    compute_out(1)

                @pl.when(p0 == 1)
                def _():
                    compute_out(0)

    return pl.pallas_call(
        body,
        out_shape=jax.ShapeDtypeStruct((B, SQ, D_MODEL), jnp.float32),
        in_specs=[pl.BlockSpec(memory_space=pltpu.VMEM)] * 5,
        out_specs=pl.BlockSpec(memory_space=pltpu.VMEM),
        scratch_shapes=[
            pltpu.VMEM((ACC_ROWS, ACC_COLS), jnp.float32),
            pltpu.VMEM((512, ACC_COLS), jnp.bfloat16),
            pltpu.VMEM((960, ACC_COLS), jnp.bfloat16),
            pltpu.VMEM((512, D_HEADS), jnp.bfloat16),
            pltpu.VMEM((960, D_HEADS), jnp.bfloat16),
            pltpu.SemaphoreType.DMA((8,)),
            pltpu.SemaphoreType.DMA((8,)),
        ],
        compiler_params=pltpu.CompilerParams(collective_id=0),
    )(x, Wq, K_ext, V_ext, Wo)


# device time: 58502 ns/iter; 9.0088x vs baseline; 1.0950x over previous
import jax
import jax.numpy as jnp
from jax import lax
from jax.experimental import pallas as pl
from jax.experimental.pallas import tpu as pltpu

N_DEV = 16
B, SQ, D_MODEL = 2, 512, 768
HQ, DH = 8, 64
SKV_LOCAL = 512
D_HEADS = HQ * DH
ACC_COLS = D_HEADS + 128
ACC_ROWS = B * SQ
RS_RECV_OFF = (0, 512, 768, 896)
AG_RECV_OFF = (0, 64, 192, 448)


def kernel(x, Wq, K_ext, V_ext, Wo):
    def body(x_ref, wq_ref, k_ref, v_ref, wo_ref, out_ref,
             acc_ref, rs_stage_ref, rs_recv_ref, ag_stage_ref, ag_recv_ref,
             send_sems, recv_sems):
        my_pos = lax.axis_index("i")

        barrier_sem = pltpu.get_barrier_semaphore()
        for k in range(4):
            pl.semaphore_signal(barrier_sem, inc=1,
                                device_id=(my_pos ^ (1 << k),),
                                device_id_type=pl.DeviceIdType.MESH)
        pl.semaphore_wait(barrier_sem, 4)

        row_iota = lax.broadcasted_iota(jnp.int32, (SQ, SKV_LOCAL), 0)
        col_iota = lax.broadcasted_iota(jnp.int32, (SQ, SKV_LOCAL), 1)
        qb = row_iota // 64
        kb = my_pos * (SKV_LOCAL // 64) + col_iota // 64
        mask = (qb == kb) | (kb == 0) | (lax.rem(qb + kb, 3) == 0)

        acc_ref[:, :] = jnp.zeros((ACC_ROWS, ACC_COLS), jnp.float32)

        def compute_partial(bt):
            xb = x_ref[pl.ds(bt, 1), :, :].reshape(SQ, D_MODEL)
            q_b = jnp.dot(xb, wq_ref[:, :],
                          preferred_element_type=jnp.float32)
            for h in range(HQ):
                q_h = q_b[:, h * DH:(h + 1) * DH]
                k_h = k_ref[pl.ds(bt, 1), :, h, :].reshape(SKV_LOCAL, DH)
                v_h = v_ref[pl.ds(bt, 1), :, h, :].reshape(SKV_LOCAL, DH)
                s = lax.dot_general(
                    q_h, k_h, (((1,), (1,)), ((), ())),
                    preferred_element_type=jnp.float32) * 0.125
                w = jnp.where(mask, jnp.exp(s), 0.0)
                l = jnp.sum(w, axis=1, keepdims=True)
                ctx = jnp.dot(w, v_h, preferred_element_type=jnp.float32)
                acc_ref[pl.ds(bt * SQ, SQ), h * DH:(h + 1) * DH] = ctx
                acc_ref[pl.ds(bt * SQ, SQ),
                        D_HEADS + h:D_HEADS + h + 1] = l

        def make_rs_rdma(send_off, half, k):
            ro = RS_RECV_OFF[k]
            rs_stage_ref[0:half, :] = acc_ref[
                pl.ds(send_off, half), :].astype(jnp.bfloat16)
            return pltpu.make_async_remote_copy(
                src_ref=rs_stage_ref.at[0:half, :],
                dst_ref=rs_recv_ref.at[ro:ro + half, :],
                send_sem=send_sems.at[k],
                recv_sem=recv_sems.at[k],
                device_id=(my_pos ^ (1 << k),),
                device_id_type=pl.DeviceIdType.MESH,
            )

        p0 = my_pos & 1
        compute_partial(1 - p0)
        rdma0 = make_rs_rdma((1 - p0) * SQ, SQ, 0)
        rdma0.start()
        compute_partial(p0)
        rdma0.wait()
        w_off = p0 * SQ
        acc_ref[pl.ds(w_off, SQ), :] = (
            acc_ref[pl.ds(w_off, SQ), :]
            + rs_recv_ref[0:SQ, :].astype(jnp.float32)
        )

        for k in range(1, 4):
            half = 512 >> k
            pk = (my_pos >> k) & 1
            send_off = w_off + (1 - pk) * half
            keep_off = w_off + pk * half
            ro = RS_RECV_OFF[k]
            rdma = make_rs_rdma(send_off, half, k)
            rdma.start()
            rdma.wait()
            acc_ref[pl.ds(keep_off, half), :] = (
                acc_ref[pl.ds(keep_off, half), :]
                + rs_recv_ref[ro:ro + half, :].astype(jnp.float32)
            )
            w_off = keep_off

        for h in range(HQ):
            ctx_h = acc_ref[pl.ds(w_off, 64), h * DH:(h + 1) * DH]
            l_h = acc_ref[pl.ds(w_off, 64), D_HEADS + h:D_HEADS + h + 1]
            acc_ref[pl.ds(w_off, 64), h * DH:(h + 1) * DH] = ctx_h / l_h

        def compute_out(bt):
            o = jnp.dot(acc_ref[pl.ds(bt * SQ, SQ), 0:D_HEADS],
                        wo_ref[:, :], preferred_element_type=jnp.float32)
            out_ref[pl.ds(bt, 1), :, :] = o.reshape(1, SQ, D_MODEL)

        a_off = w_off
        for j in range(4):
            k = 3 - j
            m = 64 << j
            pk = (my_pos >> k) & 1
            ao = AG_RECV_OFF[j]
            ag_stage_ref[0:m, :] = acc_ref[
                pl.ds(a_off, m), 0:D_HEADS].astype(jnp.bfloat16)
            rdma = pltpu.make_async_remote_copy(
                src_ref=ag_stage_ref.at[0:m, :],
                dst_ref=ag_recv_ref.at[ao:ao + m, :],
                send_sem=send_sems.at[4 + j],
                recv_sem=recv_sems.at[4 + j],
                device_id=(my_pos ^ (1 << k),),
                device_id_type=pl.DeviceIdType.MESH,
            )
            rdma.start()

            if j == 3:
                compute_out(p0)

            rdma.wait()
            sib_off = a_off + m - 2 * pk * m
            acc_ref[pl.ds(sib_off, m), 0:D_HEADS] = ag_recv_ref[
                ao:ao + m, :].astype(jnp.float32)
            a_off = a_off - pk * m

            if j == 3:
                compute_out(1 - p0)

    return pl.pallas_call(
        body,
        out_shape=jax.ShapeDtypeStruct((B, SQ, D_MODEL), jnp.float32),
        in_specs=[pl.BlockSpec(memory_space=pltpu.VMEM)] * 5,
        out_specs=pl.BlockSpec(memory_space=pltpu.VMEM),
        scratch_shapes=[
            pltpu.VMEM((ACC_ROWS, ACC_COLS), jnp.float32),
            pltpu.VMEM((512, ACC_COLS), jnp.bfloat16),
            pltpu.VMEM((960, ACC_COLS), jnp.bfloat16),
            pltpu.VMEM((512, D_HEADS), jnp.bfloat16),
            pltpu.VMEM((960, D_HEADS), jnp.bfloat16),
            pltpu.SemaphoreType.DMA((8,)),
            pltpu.SemaphoreType.DMA((8,)),
        ],
        compiler_params=pltpu.CompilerParams(collective_id=0),
    )(x, Wq, K_ext, V_ext, Wo)
